# baseline (speedup 1.0000x reference)
import jax
import jax.numpy as jnp
from jax import lax
from jax.experimental import pallas as pl
from jax.experimental.pallas import tpu as pltpu
from jax.experimental.pallas import tpu_sc as plsc

D = 768
DV = D // 16
LN_EPS = 1e-12
NW = 32
RB = 4
UNROLL = 16


def _rsqrt_scalar(var):
    i = lax.bitcast_convert_type(var, jnp.int32)
    i = jnp.int32(0x5F3759DF) - lax.shift_right_arithmetic(i, jnp.int32(1))
    y = lax.bitcast_convert_type(i, jnp.float32)
    half = var * jnp.float32(0.5)
    for _ in range(3):
        y = y * (jnp.float32(1.5) - half * y * y)
    return y


def _make_emb_ln(batch, seq):
    bp_w = batch // NW
    ng = seq
    assert batch % NW == 0 and bp_w % RB == 0

    mesh = plsc.VectorSubcoreMesh(
        core_axis_name="c", subcore_axis_name="s", num_cores=2, num_subcores=16
    )

    def body(ids_hbm, table_hbm, seg_hbm, gamma_hbm, beta_hbm, out_hbm,
             idx_v, bufs, seg_v, gamma_v, beta_v, gsem, wsem, isem):
        wid = lax.axis_index("s") * 2 + lax.axis_index("c")
        wbase = wid * bp_w
        pltpu.sync_copy(seg_hbm, seg_v)
        pltpu.sync_copy(gamma_hbm, gamma_v)
        pltpu.sync_copy(beta_hbm, beta_v)
        for k in range(3):
            pltpu.sync_copy(
                ids_hbm.at[pl.ds(k * batch + wbase, bp_w)],
                idx_v.at[pl.ds(k * bp_w, bp_w)],
            )

        def idx_copy(g, slot):
            return pltpu.make_async_copy(
                ids_hbm.at[pl.ds(g * batch + wbase, bp_w)],
                idx_v.at[pl.ds(slot * bp_w, bp_w)], isem.at[slot],
            )

        def gather_copy(slot, b):
            return pltpu.make_async_copy(
                table_hbm.at[idx_v.at[pl.ds(slot * bp_w, bp_w)]],
                bufs.at[b], gsem.at[b],
            )

        def wb_copy(g, b):
            return pltpu.make_async_copy(
                bufs.at[b], out_hbm.at[g, pl.ds(wbase, bp_w)], wsem.at[b],
            )

        gather_copy(0, 0).start()
        gather_copy(1, 1).start()

        def compute_chunk(b):
            def block(bi, _):
                r0 = bi * RB

                zero = jnp.zeros((16,), jnp.float32)

                @plsc.parallel_loop(0, DV, carry=(zero,) * (2 * RB),
                                    unroll=UNROLL)
                def p1(j, carry):
                    accs = list(carry)
                    sl = pl.ds(j * 16, 16)
                    s = seg_v[sl]
                    for r in range(RB):
                        y = bufs[b, r0 + r, sl] + s
                        bufs[b, r0 + r, sl] = y
                        accs[2 * r] = accs[2 * r] + y
                        accs[2 * r + 1] = accs[2 * r + 1] + y * y
                    return tuple(accs)

                accs = p1

                mvs, ivs = [], []
                for r in range(RB):
                    mean = jnp.sum(accs[2 * r]) * jnp.float32(1.0 / D)
                    var = (jnp.sum(accs[2 * r + 1]) * jnp.float32(1.0 / D)
                           - mean * mean)
                    inv = _rsqrt_scalar(var + jnp.float32(LN_EPS))
                    ivs.append(lax.broadcast(inv, (16,)))
                    mvs.append(lax.broadcast(mean, (16,)))

                @plsc.parallel_loop(0, DV, unroll=UNROLL)
                def p2(j):
                    sl = pl.ds(j * 16, 16)
                    gj = gamma_v[sl]
                    bj = beta_v[sl]
                    for r in range(RB):
                        y = bufs[b, r0 + r, sl]
                        bufs[b, r0 + r, sl] = (y - mvs[r]) * ivs[r] * gj + bj

                return 0

            lax.fori_loop(0, bp_w // RB, block, 0)

        def step(g, _):
            b = lax.rem(g, 3)
            b2 = lax.rem(g + 2, 3)
            gather_copy(b, b).wait()

            @pl.when(g + 3 < ng)
            def _():
                idx_copy(g + 3, b).start()

            compute_chunk(b)
            wb_copy(g, b).start()

            @pl.when(g + 2 < ng)
            def _():
                @pl.when(g >= 1)
                def _():
                    wb_copy(g - 1, b2).wait()
                    idx_copy(g + 2, b2).wait()

                gather_copy(b2, b2).start()

            return 0

        lax.fori_loop(0, ng, step, 0)

        for k in range(3):
            g = ng - 3 + k
            wb_copy(g, g % 3).wait()

    return pl.kernel(
        body,
        out_type=jax.ShapeDtypeStruct((seq, batch, D), jnp.float32),
        mesh=mesh,
        compiler_params=pltpu.CompilerParams(
            needs_layout_passes=False, use_tc_tiling_on_sc=True),
        scratch_types=[
            pltpu.VMEM((3 * bp_w,), jnp.int32),
            pltpu.VMEM((3, bp_w, D), jnp.float32),
            pltpu.VMEM((D,), jnp.float32),
            pltpu.VMEM((D,), jnp.float32),
            pltpu.VMEM((D,), jnp.float32),
            pltpu.SemaphoreType.DMA((3,)),
            pltpu.SemaphoreType.DMA((3,)),
            pltpu.SemaphoreType.DMA((3,)),
        ],
    )


def kernel(input_ids, word_table, segment_table, ln_gamma, ln_beta):
    b, s = input_ids.shape
    ids_sm = input_ids.astype(jnp.int32).T.reshape(s * b)
    out = _make_emb_ln(b, s)(
        ids_sm, word_table, segment_table[0], ln_gamma, ln_beta
    )
    return out.transpose(1, 0, 2), jnp.zeros_like(input_ids)


# speedup vs baseline: 1.6295x; 1.6295x over previous
import jax
import jax.numpy as jnp
from jax import lax
from jax.experimental import pallas as pl
from jax.experimental.pallas import tpu as pltpu
from jax.experimental.pallas import tpu_sc as plsc

D = 768
DV = D // 16
LN_EPS = 1e-12
NW = 32
RB = 4
UNROLL = 4


def _rsqrt_scalar(var):
    i = lax.bitcast_convert_type(var, jnp.int32)
    i = jnp.int32(0x5F3759DF) - lax.shift_right_arithmetic(i, jnp.int32(1))
    y = lax.bitcast_convert_type(i, jnp.float32)
    half = var * jnp.float32(0.5)
    for _ in range(3):
        y = y * (jnp.float32(1.5) - half * y * y)
    return y


def _make_emb_ln(batch, seq):
    bp_w = batch // NW
    ng = seq
    assert batch % NW == 0 and bp_w % RB == 0

    mesh = plsc.VectorSubcoreMesh(
        core_axis_name="c", subcore_axis_name="s", num_cores=2, num_subcores=16
    )

    def body(ids_hbm, table_hbm, seg_hbm, gamma_hbm, beta_hbm, out_hbm,
             idx_v, bufs, seg_v, gamma_v, beta_v, gsem, wsem, isem):
        wid = lax.axis_index("s") * 2 + lax.axis_index("c")
        wbase = wid * bp_w
        pltpu.sync_copy(seg_hbm, seg_v)
        pltpu.sync_copy(gamma_hbm, gamma_v)
        pltpu.sync_copy(beta_hbm, beta_v)
        for k in range(3):
            pltpu.sync_copy(
                ids_hbm.at[pl.ds(k * batch + wbase, bp_w)],
                idx_v.at[pl.ds(k * bp_w, bp_w)],
            )

        def idx_copy(g, slot):
            return pltpu.make_async_copy(
                ids_hbm.at[pl.ds(g * batch + wbase, bp_w)],
                idx_v.at[pl.ds(slot * bp_w, bp_w)], isem.at[slot],
            )

        def gather_copy(slot, b):
            return pltpu.make_async_copy(
                table_hbm.at[idx_v.at[pl.ds(slot * bp_w, bp_w)]],
                bufs.at[b], gsem.at[b],
            )

        def wb_copy(g, b):
            return pltpu.make_async_copy(
                bufs.at[b], out_hbm.at[g, pl.ds(wbase, bp_w)], wsem.at[b],
            )

        gather_copy(0, 0).start()
        gather_copy(1, 1).start()

        def compute_chunk(b):
            def block(bi, _):
                r0 = bi * RB

                zero = jnp.zeros((16,), jnp.float32)

                @plsc.parallel_loop(0, DV, carry=(zero,) * (2 * RB),
                                    unroll=UNROLL)
                def p1(j, carry):
                    accs = list(carry)
                    sl = pl.ds(j * 16, 16)
                    s = seg_v[sl]
                    for r in range(RB):
                        y = bufs[b, r0 + r, sl] + s
                        bufs[b, r0 + r, sl] = y
                        accs[2 * r] = accs[2 * r] + y
                        accs[2 * r + 1] = accs[2 * r + 1] + y * y
                    return tuple(accs)

                accs = p1

                mvs, ivs = [], []
                for r in range(RB):
                    mean = jnp.sum(accs[2 * r]) * jnp.float32(1.0 / D)
                    var = (jnp.sum(accs[2 * r + 1]) * jnp.float32(1.0 / D)
                           - mean * mean)
                    inv = _rsqrt_scalar(var + jnp.float32(LN_EPS))
                    ivs.append(lax.broadcast(inv, (16,)))
                    mvs.append(lax.broadcast(mean, (16,)))

                @plsc.parallel_loop(0, DV, unroll=UNROLL)
                def p2(j):
                    sl = pl.ds(j * 16, 16)
                    gj = gamma_v[sl]
                    bj = beta_v[sl]
                    for r in range(RB):
                        y = bufs[b, r0 + r, sl]
                        bufs[b, r0 + r, sl] = (y - mvs[r]) * ivs[r] * gj + bj

                return 0

            lax.fori_loop(0, bp_w // RB, block, 0)

        def step(g, _):
            b = lax.rem(g, 3)
            b2 = lax.rem(g + 2, 3)
            gather_copy(b, b).wait()

            @pl.when(g + 3 < ng)
            def _():
                idx_copy(g + 3, b).start()

            compute_chunk(b)
            wb_copy(g, b).start()

            @pl.when(g + 2 < ng)
            def _():
                @pl.when(g >= 1)
                def _():
                    wb_copy(g - 1, b2).wait()
                    idx_copy(g + 2, b2).wait()

                gather_copy(b2, b2).start()

            return 0

        lax.fori_loop(0, ng, step, 0)

        for k in range(3):
            g = ng - 3 + k
            wb_copy(g, g % 3).wait()

    return pl.kernel(
        body,
        out_type=jax.ShapeDtypeStruct((seq, batch, D), jnp.float32),
        mesh=mesh,
        compiler_params=pltpu.CompilerParams(
            needs_layout_passes=False, use_tc_tiling_on_sc=True),
        scratch_types=[
            pltpu.VMEM((3 * bp_w,), jnp.int32),
            pltpu.VMEM((3, bp_w, D), jnp.float32),
            pltpu.VMEM((D,), jnp.float32),
            pltpu.VMEM((D,), jnp.float32),
            pltpu.VMEM((D,), jnp.float32),
            pltpu.SemaphoreType.DMA((3,)),
            pltpu.SemaphoreType.DMA((3,)),
            pltpu.SemaphoreType.DMA((3,)),
        ],
    )


def kernel(input_ids, word_table, segment_table, ln_gamma, ln_beta):
    b, s = input_ids.shape
    ids_sm = input_ids.astype(jnp.int32).T.reshape(s * b)
    out = _make_emb_ln(b, s)(
        ids_sm, word_table, segment_table[0], ln_gamma, ln_beta
    )
    return out.transpose(1, 0, 2), jnp.zeros_like(input_ids)
